# baseline (speedup 1.0000x reference)
import jax
import jax.numpy as jnp
from jax import lax
from jax.experimental import pallas as pl
from jax.experimental.pallas import tpu as pltpu
from jax.experimental.pallas import tpu_sc as plsc

B, S, D = 4, 8192, 1024
_NC, _NS, _L = 2, 16, 16
_NW = _NC * _NS
_ROWS_PER_W = S // _NW
_CHUNK_ROWS = 32
_NCHUNK = _ROWS_PER_W // _CHUNK_ROWS
_CW = _CHUNK_ROWS * D


def _sc_body(x_hbm, pos_hbm, out_hbm, pos_buf, x_buf):
    wid = lax.axis_index("s") * _NC + lax.axis_index("c")
    base = wid * (_ROWS_PER_W * D)

    def add_loop(i, _):
        o = i * (8 * _L)
        for k in range(8):
            sl = pl.ds(o + k * _L, _L)
            x_buf[sl] = x_buf[sl] + pos_buf[sl]
        return 0

    for ci in range(_NCHUNK):
        off = base + ci * _CW
        pltpu.sync_copy(pos_hbm.at[pl.ds(off, _CW)], pos_buf)
        for b in range(B):
            pltpu.sync_copy(x_hbm.at[pl.ds(b * (S * D) + off, _CW)], x_buf)
            lax.fori_loop(0, _CW // (8 * _L), add_loop, 0)
            pltpu.sync_copy(x_buf, out_hbm.at[pl.ds(b * (S * D) + off, _CW)])


def _sc_kernel(x, pos_table):
    x_flat = x.reshape(B * S * D)
    pos_flat = pos_table.reshape(S * D)
    mesh = plsc.VectorSubcoreMesh(core_axis_name="c", subcore_axis_name="s")
    out = pl.kernel(
        _sc_body,
        mesh=mesh,
        out_type=jax.ShapeDtypeStruct((B * S * D,), jnp.float32),
        scratch_types=[
            pltpu.VMEM((_CW,), jnp.float32),
            pltpu.VMEM((_CW,), jnp.float32),
        ],
    )(x_flat, pos_flat)
    return out.reshape(B, S, D)


def kernel(x, pos_table):
    return _sc_kernel(x, pos_table)


# speedup vs baseline: 1.2126x; 1.2126x over previous
import jax
import jax.numpy as jnp
from jax import lax
from jax.experimental import pallas as pl
from jax.experimental.pallas import tpu as pltpu
from jax.experimental.pallas import tpu_sc as plsc

B, S, D = 4, 8192, 1024
_NC, _NS, _L = 2, 16, 16
_NW = _NC * _NS
_ROWS_PER_W = S // _NW
_CHUNK_ROWS = 16
_NCHUNK = _ROWS_PER_W // _CHUNK_ROWS
_CW = _CHUNK_ROWS * D
_NU = _NCHUNK * B


def _sc_body(x_hbm, pos_hbm, out_hbm,
             in0, in1, ou0, ou1, po0, po1,
             si0, si1, so0, so1, sp0, sp1):
    ins, outs, poss = [in0, in1], [ou0, ou1], [po0, po1]
    sins, souts, sps = [si0, si1], [so0, so1], [sp0, sp1]
    wid = lax.axis_index("s") * _NC + lax.axis_index("c")
    base = wid * (_ROWS_PER_W * D)

    def x_slice(u):
        ci, b = divmod(u, B)
        return pl.ds(b * (S * D) + base + ci * _CW, _CW)

    def start_in(u):
        return pltpu.async_copy(x_hbm.at[x_slice(u)], ins[u % 2], sins[u % 2])

    def start_pos(ci):
        return pltpu.async_copy(
            pos_hbm.at[pl.ds(base + ci * _CW, _CW)], poss[ci % 2], sps[ci % 2])

    def add_chunk(inb, posb, outb):
        def f(i, _):
            o = i * (8 * _L)
            for k in range(8):
                sl = pl.ds(o + k * _L, _L)
                outb[sl] = inb[sl] + posb[sl]
            return 0
        lax.fori_loop(0, _CW // (8 * _L), f, 0)

    hpos = {0: start_pos(0), 1: start_pos(1)}
    hin = {0: start_in(0), 1: start_in(1)}
    hout = {}
    for u in range(_NU):
        ci, b = divmod(u, B)
        pi = u % 2
        if b == 0:
            hpos.pop(ci).wait()
        hin.pop(u).wait()
        if u - 2 in hout:
            hout.pop(u - 2).wait()
        add_chunk(ins[pi], poss[ci % 2], outs[pi])
        hout[u] = pltpu.async_copy(outs[pi], out_hbm.at[x_slice(u)], souts[pi])
        if u + 2 < _NU:
            hin[u + 2] = start_in(u + 2)
        if b == B - 1 and ci + 2 < _NCHUNK:
            hpos[ci + 2] = start_pos(ci + 2)
    for u, h in sorted(hout.items()):
        h.wait()


def _sc_kernel(x, pos_table):
    x_flat = x.reshape(B * S * D)
    pos_flat = pos_table.reshape(S * D)
    mesh = plsc.VectorSubcoreMesh(core_axis_name="c", subcore_axis_name="s")
    out = pl.kernel(
        _sc_body,
        mesh=mesh,
        out_type=jax.ShapeDtypeStruct((B * S * D,), jnp.float32),
        scratch_types=(
            [pltpu.VMEM((_CW,), jnp.float32)] * 6
            + [pltpu.SemaphoreType.DMA] * 6
        ),
    )(x_flat, pos_flat)
    return out.reshape(B, S, D)


def kernel(x, pos_table):
    return _sc_kernel(x, pos_table)


# speedup vs baseline: 3.4428x; 2.8392x over previous
import jax
import jax.numpy as jnp
from jax import lax
from jax.experimental import pallas as pl
from jax.experimental.pallas import tpu as pltpu
from jax.experimental.pallas import tpu_sc as plsc

B, S, D = 4, 8192, 1024
_NC, _NS, _L = 2, 16, 16
_NW = _NC * _NS
_ROWS_PER_W = S // _NW
_CHUNK_ROWS = 16
_NCHUNK = _ROWS_PER_W // _CHUNK_ROWS
_NU = _NCHUNK * B


def _sc_body(x_hbm, pos_hbm, out_hbm,
             in0, in1, ou0, ou1, po0, po1,
             si0, si1, so0, so1, sp0, sp1):
    ins, outs, poss = [in0, in1], [ou0, ou1], [po0, po1]
    sins, souts, sps = [si0, si1], [so0, so1], [sp0, sp1]
    wid = lax.axis_index("s") * _NC + lax.axis_index("c")
    row_base = wid * _ROWS_PER_W

    def rows(ci):
        return pl.ds(row_base + ci * _CHUNK_ROWS, _CHUNK_ROWS)

    def start_in(u):
        ci, b = divmod(u, B)
        return pltpu.async_copy(x_hbm.at[b, rows(ci)], ins[u % 2], sins[u % 2])

    def start_pos(ci):
        return pltpu.async_copy(pos_hbm.at[rows(ci)], poss[ci % 2], sps[ci % 2])

    def add_chunk(inb, posb, outb):
        def f(i, _):
            r = i >> 3
            cb = (i & 7) * 128
            for k in range(8):
                sl = pl.ds(cb + k * _L, _L)
                outb[r, sl] = inb[r, sl] + posb[r, sl]
            return 0
        lax.fori_loop(0, _CHUNK_ROWS * 8, f, 0)

    hpos = {0: start_pos(0), 1: start_pos(1)}
    hin = {0: start_in(0), 1: start_in(1)}
    hout = {}
    for u in range(_NU):
        ci, b = divmod(u, B)
        pi = u % 2
        if b == 0:
            hpos.pop(ci).wait()
        hin.pop(u).wait()
        if u - 2 in hout:
            hout.pop(u - 2).wait()
        add_chunk(ins[pi], poss[ci % 2], outs[pi])
        hout[u] = pltpu.async_copy(outs[pi], out_hbm.at[b, rows(ci)], souts[pi])
        if u + 2 < _NU:
            hin[u + 2] = start_in(u + 2)
        if b == B - 1 and ci + 2 < _NCHUNK:
            hpos[ci + 2] = start_pos(ci + 2)
    for u, h in sorted(hout.items()):
        h.wait()


def _sc_kernel(x, pos_table):
    mesh = plsc.VectorSubcoreMesh(core_axis_name="c", subcore_axis_name="s")
    buf = pltpu.VMEM((_CHUNK_ROWS, D), jnp.float32)
    return pl.kernel(
        _sc_body,
        mesh=mesh,
        out_type=jax.ShapeDtypeStruct((B, S, D), jnp.float32),
        scratch_types=[buf] * 6 + [pltpu.SemaphoreType.DMA] * 6,
        compiler_params=pltpu.CompilerParams(use_tc_tiling_on_sc=True),
    )(x, pos_table)


def kernel(x, pos_table):
    return _sc_kernel(x, pos_table)
